# Initial kernel scaffold; baseline (speedup 1.0000x reference)
import jax
import jax.numpy as jnp
from jax.experimental import pallas as pl


def kernel(x, batch):
    raise NotImplementedError("write your pallas kernel here")


# speedup vs baseline: 2.5880x; 2.5880x over previous
import jax
import jax.numpy as jnp
from jax import lax
from jax.experimental import pallas as pl
from jax.experimental.pallas import tpu as pltpu
from jax.experimental.pallas import tpu_sc as plsc

D = 256
H = 128
G = 512
B = 16
CHUNK = 128
NT = 16
SEG_T = G // NT


W = 64


def _sc_sums(x_hbm, b_hbm, out_hbm,
             acc_sh,
             idx_v, idxb_v, rows_v, rb_v, ostage_v,
             idx_t, idxb_t, rows_t):
    n = x_hbm.shape[0]
    n_full = n // CHUNK
    tail = n - n_full * CHUNK
    c = lax.axis_index("c")
    s = lax.axis_index("s")

    zero16 = jnp.zeros((16,), jnp.float32)
    lane16 = lax.iota(jnp.int32, 16)
    trips = (n_full + NT - 1) // NT
    g0 = SEG_T * s

    for p in range(H // W):
        col0 = H * c + W * p

        def zrow(r, carry):
            for j in range(W // 16):
                rows_v[r, pl.ds(16 * j, 16)] = zero16
            return carry

        lax.fori_loop(0, CHUNK, zrow, 0)
        for k in range(B * G // (NT * CHUNK)):
            base = (B * G // NT) * s + CHUNK * k
            pltpu.sync_copy(rows_v, acc_sh.at[pl.ds(base, CHUNK)])
        plsc.subcore_barrier()

        def body(t, carry):
            chunk = s + NT * t

            @pl.when(chunk < n_full)
            def _():
                base = chunk * CHUNK
                pltpu.sync_copy(b_hbm.at[pl.ds(base, CHUNK)], idx_v)
                pltpu.sync_copy(
                    x_hbm.at[pl.ds(base, CHUNK), pl.ds(col0, W)], rows_v)
                for j in range(CHUNK // 16):
                    idxb_v[pl.ds(16 * j, 16)] = (
                        idx_v[pl.ds(16 * j, 16)] * B + lane16)
                pltpu.sync_copy(rows_v, acc_sh.at[idxb_v], add=True)
            return carry

        lax.fori_loop(0, trips, body, 0)

        if tail:
            @pl.when(s == NT - 1)
            def _():
                base = n_full * CHUNK
                pltpu.sync_copy(b_hbm.at[pl.ds(base, tail)], idx_t)
                pltpu.sync_copy(
                    x_hbm.at[pl.ds(base, tail), pl.ds(col0, W)], rows_t)
                for j in range(tail // 16):
                    idxb_t[pl.ds(16 * j, 16)] = (
                        idx_t[pl.ds(16 * j, 16)] * B + lane16)
                pltpu.sync_copy(rows_t, acc_sh.at[idxb_t], add=True)

        plsc.subcore_barrier()

        pltpu.sync_copy(acc_sh.at[pl.ds(B * g0, B * SEG_T)], rb_v)
        plsc.subcore_barrier()

        def reduce_one(g, carry):
            for j in range(W // 16):
                acc = rb_v[B * g, pl.ds(16 * j, 16)]
                for b in range(1, B):
                    acc = acc + rb_v[B * g + b, pl.ds(16 * j, 16)]
                ostage_v[g, pl.ds(16 * j, 16)] = acc
            return carry

        lax.fori_loop(0, SEG_T, reduce_one, 0)
        pltpu.sync_copy(ostage_v,
                        out_hbm.at[pl.ds(g0, SEG_T), pl.ds(col0, W)])


def _sc_counts(b_hbm, out_hbm,
               cnt_sh,
               idx_v, idxb_v, ones_v, zcnt_v, rbc_v, cstage_v,
               idx_t, idxb_t, ones_t):
    n = b_hbm.shape[0]
    n_full = n // CHUNK
    tail = n - n_full * CHUNK
    c = lax.axis_index("c")
    s = lax.axis_index("s")

    zero16 = jnp.zeros((16,), jnp.float32)
    one16 = jnp.ones((16,), jnp.float32)
    lane16 = lax.iota(jnp.int32, 16)

    for r in range(64):
        zcnt_v[r, :] = zero16
    for k in range(B * G // (NT * 64)):
        base = (B * G // NT) * s + 64 * k
        pltpu.sync_copy(zcnt_v, cnt_sh.at[pl.ds(base, 64)])
    for r in range(CHUNK):
        ones_v[r, :] = one16
    if tail:
        for r in range(tail):
            ones_t[r, :] = one16
    plsc.subcore_barrier()

    trips = (n_full + NT - 1) // NT

    def body(t, carry):
        chunk = s + NT * t

        @pl.when(chunk < n_full)
        def _():
            base = chunk * CHUNK
            pltpu.sync_copy(b_hbm.at[pl.ds(base, CHUNK)], idx_v)
            for j in range(CHUNK // 16):
                idxb_v[pl.ds(16 * j, 16)] = (
                    idx_v[pl.ds(16 * j, 16)] * B + lane16)
            pltpu.sync_copy(ones_v, cnt_sh.at[idxb_v], add=True)
        return carry

    lax.fori_loop(0, trips, body, 0)

    if tail:
        @pl.when(s == NT - 1)
        def _():
            base = n_full * CHUNK
            pltpu.sync_copy(b_hbm.at[pl.ds(base, tail)], idx_t)
            for j in range(tail // 16):
                idxb_t[pl.ds(16 * j, 16)] = (
                    idx_t[pl.ds(16 * j, 16)] * B + lane16)
            pltpu.sync_copy(ones_t, cnt_sh.at[idxb_t], add=True)

    plsc.subcore_barrier()

    @pl.when(c == 0)
    def _():
        g0 = SEG_T * s
        pltpu.sync_copy(cnt_sh.at[pl.ds(B * g0, B * SEG_T)], rbc_v)

        def reduce_one(g, carry):
            acc = rbc_v[B * g, :]
            for b in range(1, B):
                acc = acc + rbc_v[B * g + b, :]
            cstage_v[g, :] = acc
            return carry

        lax.fori_loop(0, SEG_T, reduce_one, 0)
        pltpu.sync_copy(cstage_v, out_hbm.at[pl.ds(g0, SEG_T)])


def _combine(sums_ref, cnt_ref, o_ref):
    inv = 1.0 / jnp.maximum(cnt_ref[:, 0:1], 1.0)
    o_ref[...] = sums_ref[...] * inv


def kernel(x, batch):
    n, d = x.shape
    assert d == D
    b32 = batch.astype(jnp.int32)

    tail = n - (n // CHUNK) * CHUNK
    tl = max(tail, 16)
    mesh = plsc.VectorSubcoreMesh(core_axis_name="c", subcore_axis_name="s")

    sums = pl.kernel(
        _sc_sums,
        out_type=jax.ShapeDtypeStruct((G, D), jnp.float32),
        mesh=mesh,
        scratch_types=[
            pltpu.VMEM_SHARED((B * G, W), jnp.float32),
            pltpu.VMEM((CHUNK,), jnp.int32),
            pltpu.VMEM((CHUNK,), jnp.int32),
            pltpu.VMEM((CHUNK, W), jnp.float32),
            pltpu.VMEM((B * SEG_T, W), jnp.float32),
            pltpu.VMEM((SEG_T, W), jnp.float32),
            pltpu.VMEM((tl,), jnp.int32),
            pltpu.VMEM((tl,), jnp.int32),
            pltpu.VMEM((tl, W), jnp.float32),
        ],
        compiler_params=pltpu.CompilerParams(use_tc_tiling_on_sc=False),
    )(x, b32)

    cnts = pl.kernel(
        _sc_counts,
        out_type=jax.ShapeDtypeStruct((G, 16), jnp.float32),
        mesh=mesh,
        scratch_types=[
            pltpu.VMEM_SHARED((B * G, 16), jnp.float32),
            pltpu.VMEM((CHUNK,), jnp.int32),
            pltpu.VMEM((CHUNK,), jnp.int32),
            pltpu.VMEM((CHUNK, 16), jnp.float32),
            pltpu.VMEM((64, 16), jnp.float32),
            pltpu.VMEM((B * SEG_T, 16), jnp.float32),
            pltpu.VMEM((SEG_T, 16), jnp.float32),
            pltpu.VMEM((tl,), jnp.int32),
            pltpu.VMEM((tl,), jnp.int32),
            pltpu.VMEM((tl, 16), jnp.float32),
        ],
        compiler_params=pltpu.CompilerParams(use_tc_tiling_on_sc=False),
    )(b32)

    out = pl.pallas_call(
        _combine,
        out_shape=jax.ShapeDtypeStruct((G, D), jnp.float32),
    )(sums, cnts)
    return out


# speedup vs baseline: 2.8412x; 1.0979x over previous
import jax
import jax.numpy as jnp
from jax import lax
from jax.experimental import pallas as pl
from jax.experimental.pallas import tpu as pltpu
from jax.experimental.pallas import tpu_sc as plsc

D = 256
H = 128
W = 64
G = 512
B = 16
CHUNK = 128
NT = 16
SEG_T = G // NT


def _sc_pool(x_hbm, b_hbm, out_hbm,
             acc_sh, cnt_sh,
             idx_v, idxb_v, rows_v, ones_v, zcnt_v,
             rb_v, rbc_v, inv_v, ostage_v,
             idx_t, idxb_t, rows_t, ones_t):
    n = x_hbm.shape[0]
    n_full = n // CHUNK
    tail = n - n_full * CHUNK
    c = lax.axis_index("c")
    s = lax.axis_index("s")

    zero16 = jnp.zeros((16,), jnp.float32)
    one16 = jnp.ones((16,), jnp.float32)
    lane16 = lax.iota(jnp.int32, 16)
    trips = (n_full + NT - 1) // NT
    g0 = SEG_T * s

    for p in range(H // W):
        col0 = H * c + W * p

        def zrow(r, carry):
            for j in range(W // 16):
                rows_v[r, pl.ds(16 * j, 16)] = zero16
            return carry

        lax.fori_loop(0, CHUNK, zrow, 0)
        for k in range(B * G // (NT * CHUNK)):
            base = (B * G // NT) * s + CHUNK * k
            pltpu.sync_copy(rows_v, acc_sh.at[pl.ds(base, CHUNK)])
        if p == 0:
            for r in range(64):
                zcnt_v[r, :] = zero16
            for k in range(B * G // (NT * 64)):
                base = (B * G // NT) * s + 64 * k
                pltpu.sync_copy(zcnt_v, cnt_sh.at[pl.ds(base, 64)])
            for r in range(CHUNK):
                ones_v[r, :] = one16
            if tail:
                for r in range(tail):
                    ones_t[r, :] = one16
        plsc.subcore_barrier()

        def body(t, carry):
            chunk = s + NT * t

            @pl.when(chunk < n_full)
            def _():
                base = chunk * CHUNK
                pltpu.sync_copy(b_hbm.at[pl.ds(base, CHUNK)], idx_v)
                pltpu.sync_copy(
                    x_hbm.at[pl.ds(base, CHUNK), pl.ds(col0, W)], rows_v)
                for j in range(CHUNK // 16):
                    idxb_v[pl.ds(16 * j, 16)] = (
                        idx_v[pl.ds(16 * j, 16)] * B + lane16)
                pltpu.sync_copy(rows_v, acc_sh.at[idxb_v], add=True)
                if p == 0:
                    pltpu.sync_copy(ones_v, cnt_sh.at[idxb_v], add=True)
            return carry

        lax.fori_loop(0, trips, body, 0)

        if tail:
            @pl.when(s == NT - 1)
            def _():
                base = n_full * CHUNK
                pltpu.sync_copy(b_hbm.at[pl.ds(base, tail)], idx_t)
                pltpu.sync_copy(
                    x_hbm.at[pl.ds(base, tail), pl.ds(col0, W)], rows_t)
                for j in range(tail // 16):
                    idxb_t[pl.ds(16 * j, 16)] = (
                        idx_t[pl.ds(16 * j, 16)] * B + lane16)
                pltpu.sync_copy(rows_t, acc_sh.at[idxb_t], add=True)
                if p == 0:
                    pltpu.sync_copy(ones_t, cnt_sh.at[idxb_t], add=True)

        plsc.subcore_barrier()

        pltpu.sync_copy(acc_sh.at[pl.ds(B * g0, B * SEG_T)], rb_v)
        if p == 0:
            pltpu.sync_copy(cnt_sh.at[pl.ds(B * g0, B * SEG_T)], rbc_v)
        plsc.subcore_barrier()

        if p == 0:
            def inv_one(g, carry):
                cb = rbc_v[B * g, :]
                for b in range(1, B):
                    cb = cb + rbc_v[B * g + b, :]
                inv_v[g, :] = 1.0 / jnp.maximum(cb, 1.0)
                return carry

            lax.fori_loop(0, SEG_T, inv_one, 0)

        def reduce_one(g, carry):
            inv = inv_v[g, :]
            for j in range(W // 16):
                acc = rb_v[B * g, pl.ds(16 * j, 16)]
                for b in range(1, B):
                    acc = acc + rb_v[B * g + b, pl.ds(16 * j, 16)]
                ostage_v[g, pl.ds(16 * j, 16)] = acc * inv
            return carry

        lax.fori_loop(0, SEG_T, reduce_one, 0)
        pltpu.sync_copy(ostage_v,
                        out_hbm.at[pl.ds(g0, SEG_T), pl.ds(col0, W)])


def kernel(x, batch):
    n, d = x.shape
    assert d == D
    b32 = batch.astype(jnp.int32)

    tail = n - (n // CHUNK) * CHUNK
    tl = max(tail, 16)
    mesh = plsc.VectorSubcoreMesh(core_axis_name="c", subcore_axis_name="s")

    out = pl.kernel(
        _sc_pool,
        out_type=jax.ShapeDtypeStruct((G, D), jnp.float32),
        mesh=mesh,
        scratch_types=[
            pltpu.VMEM_SHARED((B * G, W), jnp.float32),
            pltpu.VMEM_SHARED((B * G, 16), jnp.float32),
            pltpu.VMEM((CHUNK,), jnp.int32),
            pltpu.VMEM((CHUNK,), jnp.int32),
            pltpu.VMEM((CHUNK, W), jnp.float32),
            pltpu.VMEM((CHUNK, 16), jnp.float32),
            pltpu.VMEM((64, 16), jnp.float32),
            pltpu.VMEM((B * SEG_T, W), jnp.float32),
            pltpu.VMEM((B * SEG_T, 16), jnp.float32),
            pltpu.VMEM((SEG_T, 16), jnp.float32),
            pltpu.VMEM((SEG_T, W), jnp.float32),
            pltpu.VMEM((tl,), jnp.int32),
            pltpu.VMEM((tl,), jnp.int32),
            pltpu.VMEM((tl, W), jnp.float32),
            pltpu.VMEM((tl, 16), jnp.float32),
        ],
        compiler_params=pltpu.CompilerParams(use_tc_tiling_on_sc=False),
    )(x, b32)
    return out


# speedup vs baseline: 3.8000x; 1.3374x over previous
import jax
import jax.numpy as jnp
from jax import lax
from jax.experimental import pallas as pl
from jax.experimental.pallas import tpu as pltpu
from jax.experimental.pallas import tpu_sc as plsc

D = 256
H = 128
W = 64
G = 512
B = 16
CHUNK = 128
NT = 16
SEG_T = G // NT


def _sc_pool(x_hbm, b_hbm, bt_hbm, out_hbm,
             acc_sh, cnt_sh,
             ids_v, idxb_v, rows0_v, rows1_v, zbuf_v, ones_v, zcnt_v,
             rb_v, rbc_v, inv_v, ostage_v,
             idx_t, idxb_t, rows_t, ones_t,
             sem0, sem1):
    n = x_hbm.shape[0]
    n_full = n // CHUNK
    tail = n - n_full * CHUNK
    trips = (n_full + NT - 1) // NT
    c = lax.axis_index("c")
    s = lax.axis_index("s")

    zero16 = jnp.zeros((16,), jnp.float32)
    one16 = jnp.ones((16,), jnp.float32)
    lane16 = lax.iota(jnp.int32, 16)
    g0 = SEG_T * s
    rows = (rows0_v, rows1_v)
    sems = (sem0, sem1)

    pltpu.sync_copy(bt_hbm.at[:, pl.ds(CHUNK * s, CHUNK)], ids_v)

    def mk_idxb(t, carry):
        for j in range(CHUNK // 16):
            idxb_v[t, pl.ds(16 * j, 16)] = (
                ids_v[t, pl.ds(16 * j, 16)] * B + lane16)
        return carry

    lax.fori_loop(0, trips, mk_idxb, 0)

    def zrow(r, carry):
        for j in range(W // 16):
            zbuf_v[r, pl.ds(16 * j, 16)] = zero16
        return carry

    lax.fori_loop(0, CHUNK, zrow, 0)
    for r in range(64):
        zcnt_v[r, :] = zero16
    for r in range(CHUNK):
        ones_v[r, :] = one16
    if tail:
        for r in range(tail):
            ones_t[r, :] = one16

    for p in range(H // W):
        col0 = H * c + W * p

        for k in range(B * G // (NT * CHUNK)):
            base = (B * G // NT) * s + CHUNK * k
            pltpu.sync_copy(zbuf_v, acc_sh.at[pl.ds(base, CHUNK)])
        if p == 0:
            for k in range(B * G // (NT * 64)):
                base = (B * G // NT) * s + 64 * k
                pltpu.sync_copy(zcnt_v, cnt_sh.at[pl.ds(base, 64)])
        plsc.subcore_barrier()

        def start_load(t, slot):
            @pl.when(s + NT * t < n_full)
            def _():
                base = (s + NT * t) * CHUNK
                pltpu.async_copy(
                    x_hbm.at[pl.ds(base, CHUNK), pl.ds(col0, W)],
                    rows[slot], sems[slot])

        def wait_load(slot):
            pltpu.make_async_copy(
                x_hbm.at[pl.ds(0, CHUNK), pl.ds(col0, W)],
                rows[slot], sems[slot]).wait()

        start_load(0, 0)

        def body(t2, carry):
            for b in range(2):
                t = 2 * t2 + b

                @pl.when(s + NT * t < n_full)
                def _():
                    wait_load(b)
                    start_load(t + 1, 1 - b)
                    pltpu.sync_copy(rows[b], acc_sh.at[idxb_v.at[t]],
                                    add=True)
                    if p == 0:
                        pltpu.sync_copy(ones_v, cnt_sh.at[idxb_v.at[t]],
                                        add=True)
            return carry

        lax.fori_loop(0, (trips + 1) // 2, body, 0)

        if tail:
            @pl.when(s == NT - 1)
            def _():
                base = n_full * CHUNK
                pltpu.sync_copy(b_hbm.at[pl.ds(base, tail)], idx_t)
                pltpu.sync_copy(
                    x_hbm.at[pl.ds(base, tail), pl.ds(col0, W)], rows_t)
                for j in range(tail // 16):
                    idxb_t[pl.ds(16 * j, 16)] = (
                        idx_t[pl.ds(16 * j, 16)] * B + lane16)
                pltpu.sync_copy(rows_t, acc_sh.at[idxb_t], add=True)
                if p == 0:
                    pltpu.sync_copy(ones_t, cnt_sh.at[idxb_t], add=True)

        plsc.subcore_barrier()

        pltpu.sync_copy(acc_sh.at[pl.ds(B * g0, B * SEG_T)], rb_v)
        if p == 0:
            pltpu.sync_copy(cnt_sh.at[pl.ds(B * g0, B * SEG_T)], rbc_v)
        plsc.subcore_barrier()

        if p == 0:
            def inv_one(g, carry):
                cb = rbc_v[B * g, :]
                for b in range(1, B):
                    cb = cb + rbc_v[B * g + b, :]
                inv_v[g, :] = 1.0 / jnp.maximum(cb, 1.0)
                return carry

            lax.fori_loop(0, SEG_T, inv_one, 0)

        def reduce_one(g, carry):
            inv = inv_v[g, :]
            for j in range(W // 16):
                acc = rb_v[B * g, pl.ds(16 * j, 16)]
                for b in range(1, B):
                    acc = acc + rb_v[B * g + b, pl.ds(16 * j, 16)]
                ostage_v[g, pl.ds(16 * j, 16)] = acc * inv
            return carry

        lax.fori_loop(0, SEG_T, reduce_one, 0)
        pltpu.sync_copy(ostage_v,
                        out_hbm.at[pl.ds(g0, SEG_T), pl.ds(col0, W)])


def kernel(x, batch):
    n, d = x.shape
    assert d == D
    b32 = batch.astype(jnp.int32)

    n_full = n // CHUNK
    tail = n - n_full * CHUNK
    trips = (n_full + NT - 1) // NT
    tl = max(tail, 16)
    bt = jnp.pad(b32, (0, trips * NT * CHUNK - n)).reshape(trips, NT * CHUNK)

    mesh = plsc.VectorSubcoreMesh(core_axis_name="c", subcore_axis_name="s")
    out = pl.kernel(
        _sc_pool,
        out_type=jax.ShapeDtypeStruct((G, D), jnp.float32),
        mesh=mesh,
        scratch_types=[
            pltpu.VMEM_SHARED((B * G, W), jnp.float32),
            pltpu.VMEM_SHARED((B * G, 16), jnp.float32),
            pltpu.VMEM((trips, CHUNK), jnp.int32),
            pltpu.VMEM((trips, CHUNK), jnp.int32),
            pltpu.VMEM((CHUNK, W), jnp.float32),
            pltpu.VMEM((CHUNK, W), jnp.float32),
            pltpu.VMEM((CHUNK, W), jnp.float32),
            pltpu.VMEM((CHUNK, 16), jnp.float32),
            pltpu.VMEM((64, 16), jnp.float32),
            pltpu.VMEM((B * SEG_T, W), jnp.float32),
            pltpu.VMEM((B * SEG_T, 16), jnp.float32),
            pltpu.VMEM((SEG_T, 16), jnp.float32),
            pltpu.VMEM((SEG_T, W), jnp.float32),
            pltpu.VMEM((tl,), jnp.int32),
            pltpu.VMEM((tl,), jnp.int32),
            pltpu.VMEM((tl, W), jnp.float32),
            pltpu.VMEM((tl, 16), jnp.float32),
            pltpu.SemaphoreType.DMA,
            pltpu.SemaphoreType.DMA,
        ],
        compiler_params=pltpu.CompilerParams(use_tc_tiling_on_sc=False),
    )(x, b32, bt)
    return out


# speedup vs baseline: 4.1703x; 1.0974x over previous
import jax
import jax.numpy as jnp
from jax import lax
from jax.experimental import pallas as pl
from jax.experimental.pallas import tpu as pltpu
from jax.experimental.pallas import tpu_sc as plsc

D = 256
H = 128
W = 64
G = 512
B = 16
CHUNK = 128
NT = 16
SEG_T = G // NT


NBUF = 2


def _sc_pool(x_hbm, b_hbm, bt_hbm, out_hbm,
             acc_sh, cnt_sh,
             ids_v, idxb_v, rows0_v, rows1_v,
             zbuf_v, ones_v, zcnt_v,
             rb_v, rbc_v, inv_v, ostage_v,
             idx_t, idxb_t, rows_t, ones_t,
             lsem0, lsem1, ssem0, ssem1,
             csem):
    n = x_hbm.shape[0]
    n_full = n // CHUNK
    tail = n - n_full * CHUNK
    trips = (n_full + NT - 1) // NT
    c = lax.axis_index("c")
    s = lax.axis_index("s")

    zero16 = jnp.zeros((16,), jnp.float32)
    one16 = jnp.ones((16,), jnp.float32)
    lane16 = lax.iota(jnp.int32, 16)
    g0 = SEG_T * s
    rows = (rows0_v, rows1_v)
    lsems = (lsem0, lsem1)
    ssems = (ssem0, ssem1)

    pltpu.sync_copy(bt_hbm.at[:, pl.ds(CHUNK * s, CHUNK)], ids_v)

    def mk_idxb(t, carry):
        for j in range(CHUNK // 16):
            idxb_v[t, pl.ds(16 * j, 16)] = (
                ids_v[t, pl.ds(16 * j, 16)] * B + lane16)
        return carry

    lax.fori_loop(0, trips, mk_idxb, 0)

    def zrow(r, carry):
        for j in range(W // 16):
            zbuf_v[r, pl.ds(16 * j, 16)] = zero16
        return carry

    lax.fori_loop(0, CHUNK, zrow, 0)
    for r in range(64):
        zcnt_v[r, :] = zero16
    for r in range(CHUNK):
        ones_v[r, :] = one16
    if tail:
        for r in range(tail):
            ones_t[r, :] = one16

    for p in range(H // W):
        col0 = H * c + W * p

        for k in range(B * G // (NT * CHUNK)):
            base = (B * G // NT) * s + CHUNK * k
            pltpu.sync_copy(zbuf_v, acc_sh.at[pl.ds(base, CHUNK)])
        if p == 0:
            for k in range(B * G // (NT * 64)):
                base = (B * G // NT) * s + 64 * k
                pltpu.sync_copy(zcnt_v, cnt_sh.at[pl.ds(base, 64)])
        plsc.subcore_barrier()

        def start_load(t, slot):
            @pl.when(s + NT * t < n_full)
            def _():
                base = (s + NT * t) * CHUNK
                pltpu.async_copy(
                    x_hbm.at[pl.ds(base, CHUNK), pl.ds(col0, W)],
                    rows[slot], lsems[slot])

        def wait_load(slot):
            pltpu.make_async_copy(
                x_hbm.at[pl.ds(0, CHUNK), pl.ds(col0, W)],
                rows[slot], lsems[slot]).wait()

        def wait_scatter(t, slot):
            pltpu.make_async_copy(rows[slot], acc_sh.at[idxb_v.at[t]],
                                  ssems[slot]).wait()

        for b in range(NBUF):
            start_load(b, b)

        def body(t2, carry):
            for b in range(NBUF):
                t = NBUF * t2 + b

                @pl.when(s + NT * t < n_full)
                def _():
                    wait_load(b)
                    pltpu.async_copy(rows[b], acc_sh.at[idxb_v.at[t]],
                                     ssems[b], add=True)
                    if p == 0:
                        pltpu.async_copy(ones_v, cnt_sh.at[idxb_v.at[t]],
                                         csem, add=True)

                @pl.when(s + NT * (t + NBUF) < n_full)
                def _():
                    wait_scatter(t, b)
                    start_load(t + NBUF, b)
            return carry

        lax.fori_loop(0, (trips + NBUF - 1) // NBUF, body, 0)

        v = (n_full - 1 - s) // NT + 1

        for b in range(NBUF):
            t_last = b + NBUF * ((v - 1 - b) // NBUF)

            @pl.when(v > b)
            def _():
                wait_scatter(t_last, b)

        if p == 0:
            def drain_cnt(t, carry):
                @pl.when(s + NT * t < n_full)
                def _():
                    pltpu.make_async_copy(
                        ones_v, cnt_sh.at[idxb_v.at[t]], csem).wait()
                return carry

            lax.fori_loop(0, trips, drain_cnt, 0)

        if tail:
            @pl.when(s == NT - 1)
            def _():
                base = n_full * CHUNK
                pltpu.sync_copy(b_hbm.at[pl.ds(base, tail)], idx_t)
                pltpu.sync_copy(
                    x_hbm.at[pl.ds(base, tail), pl.ds(col0, W)], rows_t)
                for j in range(tail // 16):
                    idxb_t[pl.ds(16 * j, 16)] = (
                        idx_t[pl.ds(16 * j, 16)] * B + lane16)
                pltpu.sync_copy(rows_t, acc_sh.at[idxb_t], add=True)
                if p == 0:
                    pltpu.sync_copy(ones_t, cnt_sh.at[idxb_t], add=True)

        plsc.subcore_barrier()

        pltpu.sync_copy(acc_sh.at[pl.ds(B * g0, B * SEG_T)], rb_v)
        if p == 0:
            pltpu.sync_copy(cnt_sh.at[pl.ds(B * g0, B * SEG_T)], rbc_v)
        plsc.subcore_barrier()

        if p == 0:
            def inv_one(g, carry):
                cb = rbc_v[B * g, :]
                for b in range(1, B):
                    cb = cb + rbc_v[B * g + b, :]
                inv_v[g, :] = 1.0 / jnp.maximum(cb, 1.0)
                return carry

            lax.fori_loop(0, SEG_T, inv_one, 0)

        def reduce_one(g, carry):
            inv = inv_v[g, :]
            for j in range(W // 16):
                acc = rb_v[B * g, pl.ds(16 * j, 16)]
                for b in range(1, B):
                    acc = acc + rb_v[B * g + b, pl.ds(16 * j, 16)]
                ostage_v[g, pl.ds(16 * j, 16)] = acc * inv
            return carry

        lax.fori_loop(0, SEG_T, reduce_one, 0)
        pltpu.sync_copy(ostage_v,
                        out_hbm.at[pl.ds(g0, SEG_T), pl.ds(col0, W)])


def kernel(x, batch):
    n, d = x.shape
    assert d == D
    b32 = batch.astype(jnp.int32)

    n_full = n // CHUNK
    tail = n - n_full * CHUNK
    trips = (n_full + NT - 1) // NT
    tl = max(tail, 16)
    bt = jnp.pad(b32, (0, trips * NT * CHUNK - n)).reshape(trips, NT * CHUNK)

    mesh = plsc.VectorSubcoreMesh(core_axis_name="c", subcore_axis_name="s")
    out = pl.kernel(
        _sc_pool,
        out_type=jax.ShapeDtypeStruct((G, D), jnp.float32),
        mesh=mesh,
        scratch_types=[
            pltpu.VMEM_SHARED((B * G, W), jnp.float32),
            pltpu.VMEM_SHARED((B * G, 16), jnp.float32),
            pltpu.VMEM((trips, CHUNK), jnp.int32),
            pltpu.VMEM((trips, CHUNK), jnp.int32),
            pltpu.VMEM((CHUNK, W), jnp.float32),
            pltpu.VMEM((CHUNK, W), jnp.float32),
            pltpu.VMEM((CHUNK, W), jnp.float32),
            pltpu.VMEM((CHUNK, 16), jnp.float32),
            pltpu.VMEM((64, 16), jnp.float32),
            pltpu.VMEM((B * SEG_T, W), jnp.float32),
            pltpu.VMEM((B * SEG_T, 16), jnp.float32),
            pltpu.VMEM((SEG_T, 16), jnp.float32),
            pltpu.VMEM((SEG_T, W), jnp.float32),
            pltpu.VMEM((tl,), jnp.int32),
            pltpu.VMEM((tl,), jnp.int32),
            pltpu.VMEM((tl, W), jnp.float32),
            pltpu.VMEM((tl, 16), jnp.float32),
            pltpu.SemaphoreType.DMA,
            pltpu.SemaphoreType.DMA,
            pltpu.SemaphoreType.DMA,
            pltpu.SemaphoreType.DMA,
            pltpu.SemaphoreType.DMA,
        ],
        compiler_params=pltpu.CompilerParams(use_tc_tiling_on_sc=False),
    )(x, b32, bt)
    return out
